# Initial kernel scaffold; baseline (speedup 1.0000x reference)
import jax
import jax.numpy as jnp
from jax.experimental import pallas as pl


def kernel(x, modality_table, modality_id):
    raise NotImplementedError("write your pallas kernel here")


# speedup vs baseline: 1.0038x; 1.0038x over previous
import jax
import jax.numpy as jnp
from jax.experimental import pallas as pl
from jax.experimental.pallas import tpu as pltpu


def _add_kernel(mid_ref, table_ref, x_ref, o_ref):
    row = table_ref[mid_ref[0], :]
    o_ref[...] = x_ref[...] + row[None, :]


def kernel(x, modality_table, modality_id):
    B, S, E = x.shape
    rows = B * S
    x2 = x.reshape(rows, E)
    block = 1024
    grid = rows // block
    mid = jnp.asarray(modality_id, jnp.int32).reshape((1,))
    out = pl.pallas_call(
        _add_kernel,
        grid_spec=pltpu.PrefetchScalarGridSpec(
            num_scalar_prefetch=1,
            grid=(grid,),
            in_specs=[
                pl.BlockSpec(modality_table.shape, lambda i, m: (0, 0)),
                pl.BlockSpec((block, E), lambda i, m: (i, 0)),
            ],
            out_specs=pl.BlockSpec((block, E), lambda i, m: (i, 0)),
        ),
        out_shape=jax.ShapeDtypeStruct((rows, E), x.dtype),
    )(mid, modality_table, x2)
    return out.reshape(B, S, E)
